# baseline (speedup 1.0000x reference)
import functools

import jax
import jax.numpy as jnp
from jax import lax
from jax.experimental import pallas as pl
from jax.experimental.pallas import tpu as pltpu
from jax.experimental.pallas import tpu_sc as plsc

N = 10000
E = 320000
FIN = 128
NH = 8
FOUT = 16
DF = 128
NHH = 4
NC = 2
NS = 16
EPS = E // NS
C = 32
SUP = 12
NSUP = -(-EPS // (C * SUP))
NCHUNK = NSUP * SUP
EPT_PAD = NCHUNK * C
BLK = SUP * 2 * C
ROWS_PT = 624
EPI = 16
NEG_SLOPE = 0.2

_GDN = jax.lax.GatherDimensionNumbers(
    offset_dims=(), collapsed_slice_dims=(0,), start_index_map=(0,))


def _lane_bcast(v, j):
    idx = jnp.full((16, 1), j, jnp.int32)
    return jax.lax.gather(
        v, idx, _GDN, slice_sizes=(1,),
        mode=jax.lax.GatherScatterMode.PROMISE_IN_BOUNDS)


def _tc_prologue(x, W, M):
    def body(x_ref, w_ref, m_ref, t_ref, mx_ref):
        P = jnp.dot(x_ref[...], w_ref[...], preferred_element_type=jnp.float32)
        S = jnp.dot(P, m_ref[...], preferred_element_type=jnp.float32)
        t_ref[0] = jnp.concatenate([P[:, 0:64], S[:, 0:64]], axis=1)
        t_ref[1] = jnp.concatenate([P[:, 64:128], S[:, 64:128]], axis=1)
        ssmax = jnp.maximum(jnp.max(S[:, 0:4]), jnp.max(S[:, 64:68]))
        stmax = jnp.maximum(jnp.max(S[:, 16:20]), jnp.max(S[:, 80:84]))
        mx_ref[...] = jnp.concatenate(
            [jnp.full((64,), ssmax, jnp.float32),
             jnp.full((64,), stmax, jnp.float32)])[None]

    return pl.pallas_call(
        body,
        out_shape=[
            jax.ShapeDtypeStruct((NC, N, DF), jnp.float32),
            jax.ShapeDtypeStruct((1, 128), jnp.float32),
        ],
    )(x, W, M)


def _tc_epilogue(out2, x, skip_W, bias):
    def body(o_ref, x_ref, sw_ref, b_ref, out_ref):
        skip = jnp.dot(x_ref[...], sw_ref[...],
                       preferred_element_type=jnp.float32)
        v = jnp.concatenate([o_ref[0], o_ref[1]], axis=1) + skip + b_ref[...]
        out_ref[...] = jnp.where(v > 0, v, jnp.exp(jnp.minimum(v, 0.0)) - 1.0)

    return pl.pallas_call(
        body,
        out_shape=jax.ShapeDtypeStruct((N, NH * FOUT), jnp.float32),
    )(out2, x, skip_W, bias)


def _sc_main(ttab, mxf, eidx):
    mesh = plsc.VectorSubcoreMesh(core_axis_name="c", subcore_axis_name="s")

    @functools.partial(
        pl.kernel,
        out_type=jax.ShapeDtypeStruct((NC, N, 64), jnp.float32),
        mesh=mesh,
        compiler_params=pltpu.CompilerParams(needs_layout_passes=False),
        scratch_types=[
            pltpu.VMEM((BLK,), jnp.int32),
            pltpu.VMEM((BLK,), jnp.int32),
            pltpu.VMEM((C,), jnp.int32),
            pltpu.VMEM((C,), jnp.int32),
            pltpu.VMEM((C, DF), jnp.float32),
            pltpu.VMEM((C, DF), jnp.float32),
            pltpu.VMEM((C, DF), jnp.float32),
            pltpu.VMEM((C, DF), jnp.float32),
            pltpu.VMEM((C, DF), jnp.float32),
            pltpu.VMEM((C, DF), jnp.float32),
            pltpu.VMEM((C, DF), jnp.float32),
            pltpu.VMEM((C, DF), jnp.float32),
            pltpu.VMEM((128,), jnp.float32),
            pltpu.VMEM((EPI, 64), jnp.float32),
            pltpu.VMEM_SHARED((N, DF), jnp.float32),
            pltpu.SemaphoreType.DMA,
            pltpu.SemaphoreType.DMA,
            pltpu.SemaphoreType.DMA,
            pltpu.SemaphoreType.DMA,
            pltpu.SemaphoreType.DMA,
        ],
    )
    def body(t_h, mx_h, ei_h, out_h,
             blk, abuf, ts0, ts1, pg0, pg1, pg2, qg0, qg1, qg2, ob0, ob1,
             mxb, ob2, acc, semg0, semg1, semg2, sems0, sems1):
        c = lax.axis_index("c")
        s = lax.axis_index("s")
        iota = lax.iota(jnp.int32, 16)
        nblk = jnp.where(s == NS - 1, 40, ROWS_PT // EPI)
        cN = jnp.full((16,), c * N, jnp.int32)
        ts = [ts0, ts1]
        pg = [pg0, pg1, pg2]
        qg = [qg0, qg1, qg2]
        ob = [ob0, ob1]
        semg = [semg0, semg1, semg2]
        sems = [sems0, sems1]

        pltpu.sync_copy(mx_h, mxb)

        zeros16 = jnp.zeros((16,), jnp.float32)
        for r in range(C):
            for q in range(8):
                ob0[r, pl.ds(q * 16, 16)] = zeros16
                ob1[r, pl.ds(q * 16, 16)] = zeros16

        def zloop(q, carry):
            r0 = s * ROWS_PT + q * EPI
            pltpu.sync_copy(ob0.at[pl.ds(0, EPI)], acc.at[pl.ds(r0, EPI)])
            return carry
        lax.fori_loop(0, nblk, zloop, 0)

        plsc.subcore_barrier()

        mpre = mxb[0:16] + mxb[pl.ds(64, 16)]
        mvec = jnp.maximum(mpre, NEG_SLOPE * mpre)
        mask4 = jnp.minimum(jnp.maximum(
            (4 - iota).astype(jnp.float32), 0.0), 1.0)

        def fire_gathers(j, x):
            so = j * 2 * C
            pltpu.async_copy(t_h.at[abuf.at[pl.ds(so, C)]], pg[x], semg[x])
            pltpu.async_copy(t_h.at[abuf.at[pl.ds(so + C, C)]], qg[x], semg[x])

        def wait_gathers(x):
            pltpu.make_async_copy(t_h.at[pl.ds(0, C)], pg[x], semg[x]).wait()
            pltpu.make_async_copy(t_h.at[pl.ds(0, C)], qg[x], semg[x]).wait()

        def wait_scatter(x):
            pltpu.make_async_copy(t_h.at[pl.ds(0, C)], ob[x], sems[x]).wait()

        def compute_chunk(g, gx, x):
            pgx, qgx, obx = pg[gx], qg[gx], ob[x]

            def edge(e, carry2):
                srow = pgx[e, pl.ds(64, 16)] + qgx[e, pl.ds(80, 16)]
                srow = jnp.maximum(srow, NEG_SLOPE * srow)
                w = jnp.exp(srow - mvec)
                w = w * mask4 * jnp.where(g * C + e < EPS, 1.0, 0.0)
                obx[e, pl.ds(64, 16)] = w
                for h in range(NHH):
                    wh = _lane_bcast(w, h)
                    fsl = pl.ds(h * 16, 16)
                    obx[e, fsl] = pgx[e, fsl] * wh
                return carry2
            lax.fori_loop(0, C, edge, 0, unroll=2)

        def sup(u, carry):
            pltpu.sync_copy(ei_h.at[pl.ds(s * (NCHUNK * 2 * C) + u * BLK,
                                          BLK)], blk)
            for i in range(BLK // 16):
                sl = pl.ds(i * 16, 16)
                abuf[sl] = blk[sl] + cN
            fire_gathers(0, 0)
            fire_gathers(1, 1)
            for j in range(SUP):
                gx = j % 3
                x = j % 2
                wait_gathers(gx)
                if j < SUP - 2:
                    fire_gathers(j + 2, (j + 2) % 3)
                if j < 2:
                    @pl.when(u > 0)
                    def _():
                        wait_scatter(x)
                else:
                    wait_scatter(x)
                for i in range(C // 16):
                    sl = pl.ds(i * 16, 16)
                    ts[x][sl] = blk[pl.ds(j * 2 * C + C + i * 16, 16)]
                compute_chunk(u * SUP + j, gx, x)
                pltpu.async_copy(ob[x], acc.at[ts[x]], sems[x], add=True)
            return carry
        lax.fori_loop(0, NSUP, sup, 0)

        wait_scatter(0)
        wait_scatter(1)
        plsc.subcore_barrier()

        def eloop(q, carry):
            r0 = s * ROWS_PT + q * EPI
            pltpu.sync_copy(acc.at[pl.ds(r0, EPI)], pg0.at[pl.ds(0, EPI)])
            for r in range(EPI):
                drow = pg0[r, pl.ds(64, 16)]
                for h in range(NHH):
                    dv = _lane_bcast(drow, h)
                    v = pg0[r, pl.ds(h * 16, 16)] / (dv + 1e-16)
                    ob2[r, pl.ds(h * 16, 16)] = v
            pltpu.sync_copy(ob2, out_h.at[c, pl.ds(r0, EPI)])
            return carry
        lax.fori_loop(0, nblk, eloop, 0)

    return body(ttab, mxf, eidx)


def kernel(x, edge_index, W, a_src, a_trg, skip_W, bias):
    pad = EPT_PAD - EPS
    src_c = jnp.pad(edge_index[0].reshape(NS, EPS),
                    ((0, 0), (0, pad))).reshape(NS, NCHUNK, 1, C)
    trg_c = jnp.pad(edge_index[1].reshape(NS, EPS),
                    ((0, 0), (0, pad))).reshape(NS, NCHUNK, 1, C)
    eidx = jnp.concatenate([src_c, trg_c], axis=2).reshape(-1)

    eye = jnp.eye(NH, dtype=jnp.float32)
    bds = (a_src[0][:, :, None] * eye[:, None, :]).reshape(NH * FOUT, NH)
    bdt = (a_trg[0][:, :, None] * eye[:, None, :]).reshape(NH * FOUT, NH)
    z12 = jnp.zeros((FIN, 12), jnp.float32)
    z44 = jnp.zeros((FIN, 44), jnp.float32)
    M = jnp.concatenate(
        [bds[:, 0:4], z12, bdt[:, 0:4], z44,
         bds[:, 4:8], z12, bdt[:, 4:8], z44], axis=1)

    T2, MX = _tc_prologue(x, W, M)
    out2 = _sc_main(T2.reshape(NC * N, DF), MX.reshape(128), eidx)
    return _tc_epilogue(out2, x, skip_W, bias.reshape(1, NH * FOUT))


# speedup vs baseline: 1.0200x; 1.0200x over previous
import functools

import jax
import jax.numpy as jnp
from jax import lax
from jax.experimental import pallas as pl
from jax.experimental.pallas import tpu as pltpu
from jax.experimental.pallas import tpu_sc as plsc

N = 10000
E = 320000
FIN = 128
NH = 8
FOUT = 16
DF = 128
NHH = 4
NC = 2
NS = 16
EPS = E // NS
C = 48
SUP = 8
NSUP = -(-EPS // (C * SUP))
NCHUNK = NSUP * SUP
EPT_PAD = NCHUNK * C
BLK = SUP * 2 * C
ROWS_PT = 624
EPI = 16
NEG_SLOPE = 0.2

_GDN = jax.lax.GatherDimensionNumbers(
    offset_dims=(), collapsed_slice_dims=(0,), start_index_map=(0,))


def _lane_bcast(v, j):
    idx = jnp.full((16, 1), j, jnp.int32)
    return jax.lax.gather(
        v, idx, _GDN, slice_sizes=(1,),
        mode=jax.lax.GatherScatterMode.PROMISE_IN_BOUNDS)


def _tc_prologue(x, W, M):
    def body(x_ref, w_ref, m_ref, t_ref, mx_ref):
        P = jnp.dot(x_ref[...], w_ref[...], preferred_element_type=jnp.float32)
        S = jnp.dot(P, m_ref[...], preferred_element_type=jnp.float32)
        t_ref[0] = jnp.concatenate([P[:, 0:64], S[:, 0:64]], axis=1)
        t_ref[1] = jnp.concatenate([P[:, 64:128], S[:, 64:128]], axis=1)
        ssmax = jnp.maximum(jnp.max(S[:, 0:4]), jnp.max(S[:, 64:68]))
        stmax = jnp.maximum(jnp.max(S[:, 16:20]), jnp.max(S[:, 80:84]))
        mx_ref[...] = jnp.concatenate(
            [jnp.full((64,), ssmax, jnp.float32),
             jnp.full((64,), stmax, jnp.float32)])[None]

    return pl.pallas_call(
        body,
        out_shape=[
            jax.ShapeDtypeStruct((NC, N, DF), jnp.float32),
            jax.ShapeDtypeStruct((1, 128), jnp.float32),
        ],
    )(x, W, M)


def _tc_epilogue(out2, x, skip_W, bias):
    def body(o_ref, x_ref, sw_ref, b_ref, out_ref):
        skip = jnp.dot(x_ref[...], sw_ref[...],
                       preferred_element_type=jnp.float32)
        v = jnp.concatenate([o_ref[0], o_ref[1]], axis=1) + skip + b_ref[...]
        out_ref[...] = jnp.where(v > 0, v, jnp.exp(jnp.minimum(v, 0.0)) - 1.0)

    return pl.pallas_call(
        body,
        out_shape=jax.ShapeDtypeStruct((N, NH * FOUT), jnp.float32),
    )(out2, x, skip_W, bias)


def _sc_main(ttab, mxf, eidx):
    mesh = plsc.VectorSubcoreMesh(core_axis_name="c", subcore_axis_name="s")

    @functools.partial(
        pl.kernel,
        out_type=jax.ShapeDtypeStruct((NC, N, 64), jnp.float32),
        mesh=mesh,
        compiler_params=pltpu.CompilerParams(needs_layout_passes=False),
        scratch_types=[
            pltpu.VMEM((BLK,), jnp.int32),
            pltpu.VMEM((BLK,), jnp.int32),
            pltpu.VMEM((C,), jnp.int32),
            pltpu.VMEM((C,), jnp.int32),
            pltpu.VMEM((C, DF), jnp.float32),
            pltpu.VMEM((C, DF), jnp.float32),
            pltpu.VMEM((C, DF), jnp.float32),
            pltpu.VMEM((C, DF), jnp.float32),
            pltpu.VMEM((C, DF), jnp.float32),
            pltpu.VMEM((C, DF), jnp.float32),
            pltpu.VMEM((128,), jnp.float32),
            pltpu.VMEM((EPI, 64), jnp.float32),
            pltpu.VMEM_SHARED((N, DF), jnp.float32),
            pltpu.SemaphoreType.DMA,
            pltpu.SemaphoreType.DMA,
            pltpu.SemaphoreType.DMA,
            pltpu.SemaphoreType.DMA,
        ],
    )
    def body(t_h, mx_h, ei_h, out_h,
             blk, abuf, ts0, ts1, pg0, pg1, qg0, qg1, ob0, ob1,
             mxb, ob2, acc, semg0, semg1, sems0, sems1):
        c = lax.axis_index("c")
        s = lax.axis_index("s")
        iota = lax.iota(jnp.int32, 16)
        nblk = jnp.where(s == NS - 1, 40, ROWS_PT // EPI)
        cN = jnp.full((16,), c * N, jnp.int32)
        ts = [ts0, ts1]
        pg = [pg0, pg1]
        qg = [qg0, qg1]
        ob = [ob0, ob1]
        semg = [semg0, semg1]
        sems = [sems0, sems1]

        pltpu.sync_copy(mx_h, mxb)

        zeros16 = jnp.zeros((16,), jnp.float32)
        for r in range(C):
            for q in range(8):
                ob0[r, pl.ds(q * 16, 16)] = zeros16
                ob1[r, pl.ds(q * 16, 16)] = zeros16

        def zloop(q, carry):
            r0 = s * ROWS_PT + q * EPI
            pltpu.sync_copy(ob0.at[pl.ds(0, EPI)], acc.at[pl.ds(r0, EPI)])
            return carry
        lax.fori_loop(0, nblk, zloop, 0)

        plsc.subcore_barrier()

        mpre = mxb[0:16] + mxb[pl.ds(64, 16)]
        mvec = jnp.maximum(mpre, NEG_SLOPE * mpre)
        mask4 = jnp.minimum(jnp.maximum(
            (4 - iota).astype(jnp.float32), 0.0), 1.0)

        def fire_gathers(j, x):
            so = j * 2 * C
            pltpu.async_copy(t_h.at[abuf.at[pl.ds(so, C)]], pg[x], semg[x])
            pltpu.async_copy(t_h.at[abuf.at[pl.ds(so + C, C)]], qg[x], semg[x])

        def wait_gathers(x):
            pltpu.make_async_copy(t_h.at[pl.ds(0, C)], pg[x], semg[x]).wait()
            pltpu.make_async_copy(t_h.at[pl.ds(0, C)], qg[x], semg[x]).wait()

        def wait_scatter(x):
            pltpu.make_async_copy(t_h.at[pl.ds(0, C)], ob[x], sems[x]).wait()

        def compute_chunk(g, x):
            pgx, qgx, obx = pg[x], qg[x], ob[x]

            def edge_fast(e, carry2):
                srow = pgx[e, pl.ds(64, 16)] + qgx[e, pl.ds(80, 16)]
                srow = jnp.maximum(srow, NEG_SLOPE * srow)
                w = jnp.exp(srow - mvec) * mask4
                obx[e, pl.ds(64, 16)] = w
                for h in range(NHH):
                    wh = _lane_bcast(w, h)
                    fsl = pl.ds(h * 16, 16)
                    obx[e, fsl] = pgx[e, fsl] * wh
                return carry2

            def edge_masked(e, carry2):
                srow = pgx[e, pl.ds(64, 16)] + qgx[e, pl.ds(80, 16)]
                srow = jnp.maximum(srow, NEG_SLOPE * srow)
                w = jnp.exp(srow - mvec)
                w = w * mask4 * jnp.where(g * C + e < EPS, 1.0, 0.0)
                obx[e, pl.ds(64, 16)] = w
                for h in range(NHH):
                    wh = _lane_bcast(w, h)
                    fsl = pl.ds(h * 16, 16)
                    obx[e, fsl] = pgx[e, fsl] * wh
                return carry2

            @pl.when(g < EPS // C)
            def _():
                lax.fori_loop(0, C, edge_fast, 0, unroll=4)

            @pl.when(g >= EPS // C)
            def _():
                lax.fori_loop(0, C, edge_masked, 0, unroll=2)

        def sup(u, carry):
            pltpu.sync_copy(ei_h.at[pl.ds(s * (NCHUNK * 2 * C) + u * BLK,
                                          BLK)], blk)
            for i in range(BLK // 16):
                sl = pl.ds(i * 16, 16)
                abuf[sl] = blk[sl] + cN
            fire_gathers(0, 0)
            for j in range(SUP):
                x = j % 2
                wait_gathers(x)
                if j < SUP - 1:
                    fire_gathers(j + 1, 1 - x)
                if j < 2:
                    @pl.when(u > 0)
                    def _():
                        wait_scatter(x)
                else:
                    wait_scatter(x)
                for i in range(C // 16):
                    sl = pl.ds(i * 16, 16)
                    ts[x][sl] = blk[pl.ds(j * 2 * C + C + i * 16, 16)]
                compute_chunk(u * SUP + j, x)
                pltpu.async_copy(ob[x], acc.at[ts[x]], sems[x], add=True)
            return carry
        lax.fori_loop(0, NSUP, sup, 0)

        wait_scatter(0)
        wait_scatter(1)
        plsc.subcore_barrier()

        def eloop(q, carry):
            r0 = s * ROWS_PT + q * EPI
            pltpu.sync_copy(acc.at[pl.ds(r0, EPI)], pg0.at[pl.ds(0, EPI)])
            for r in range(EPI):
                drow = pg0[r, pl.ds(64, 16)]
                for h in range(NHH):
                    dv = _lane_bcast(drow, h)
                    v = pg0[r, pl.ds(h * 16, 16)] / (dv + 1e-16)
                    ob2[r, pl.ds(h * 16, 16)] = v
            pltpu.sync_copy(ob2, out_h.at[c, pl.ds(r0, EPI)])
            return carry
        lax.fori_loop(0, nblk, eloop, 0)

    return body(ttab, mxf, eidx)


def kernel(x, edge_index, W, a_src, a_trg, skip_W, bias):
    pad = EPT_PAD - EPS
    src_c = jnp.pad(edge_index[0].reshape(NS, EPS),
                    ((0, 0), (0, pad))).reshape(NS, NCHUNK, 1, C)
    trg_c = jnp.pad(edge_index[1].reshape(NS, EPS),
                    ((0, 0), (0, pad))).reshape(NS, NCHUNK, 1, C)
    eidx = jnp.concatenate([src_c, trg_c], axis=2).reshape(-1)

    eye = jnp.eye(NH, dtype=jnp.float32)
    bds = (a_src[0][:, :, None] * eye[:, None, :]).reshape(NH * FOUT, NH)
    bdt = (a_trg[0][:, :, None] * eye[:, None, :]).reshape(NH * FOUT, NH)
    z12 = jnp.zeros((FIN, 12), jnp.float32)
    z44 = jnp.zeros((FIN, 44), jnp.float32)
    M = jnp.concatenate(
        [bds[:, 0:4], z12, bdt[:, 0:4], z44,
         bds[:, 4:8], z12, bdt[:, 4:8], z44], axis=1)

    T2, MX = _tc_prologue(x, W, M)
    out2 = _sc_main(T2.reshape(NC * N, DF), MX.reshape(128), eidx)
    return _tc_epilogue(out2, x, skip_W, bias.reshape(1, NH * FOUT))
